# baseline (device time: 59485 ns/iter reference)
import numpy as np
import jax
import jax.numpy as jnp
from jax import lax
from jax.experimental import pallas as pl
from jax.experimental.pallas import tpu as pltpu

N_DEV = 4
SQ = 1024
D = 1024
HQ = 8
DH = 128
CH = SQ // N_DEV
HD = D // 2
SCALE = 0.08838834764831843

_inv = 1.0 / (10000.0 ** (np.arange(0, DH, 2) / DH))
_pos = np.arange(SQ)[:, None] * _inv[None, :]
_COS = np.tile(np.repeat(np.cos(_pos), 2, axis=-1), (1, HQ)).astype(np.float32)
_SIN = np.tile(np.repeat(np.sin(_pos), 2, axis=-1), (1, HQ)).astype(np.float32)


def kernel(x, Wq, Wk, Wv, Wo):

    def body(x_ref, wq_ref, wk_ref, wv_ref, wo_ref, cos_ref, sin_ref,
             out_ref, xb_ref, wqb_ref, wob_ref, k_ref, v_ref, ctx_ref,
             pr_ref, pl_ref, rsr_ref, rsl_ref, agr_ref, agl_ref,
             rsr_send, rsr_recv, rsl_send, rsl_recv,
             agr_send, agr_recv, agl_send, agl_recv):
        my = lax.axis_index("i")
        left = lax.rem(my + (N_DEV - 1), N_DEV)
        right = lax.rem(my + 1, N_DEV)

        bar = pltpu.get_barrier_semaphore()
        for nbr in (left, right):
            pl.semaphore_signal(bar, inc=1, device_id=(nbr,),
                                device_id_type=pl.DeviceIdType.MESH)
        pl.semaphore_wait(bar, 2)

        def rope(t, cosr, sinr):
            n = t.shape[1]
            even = (lax.broadcasted_iota(jnp.int32, t.shape, 1) % 2) == 0
            t_next = pltpu.roll(t, n - 1, 1)
            t_prev = pltpu.roll(t, 1, 1)
            return t * cosr + jnp.where(even, -t_next, t_prev) * sinr

        cos_f = cos_ref[...]
        sin_f = sin_ref[...]

        xb_ref[...] = x_ref[0].astype(jnp.bfloat16)
        wqb_ref[...] = wq_ref[...].astype(jnp.bfloat16)
        wob_ref[...] = wo_ref[...].astype(jnp.bfloat16)
        xb = xb_ref[...]

        k_ref[...] = rope(jnp.dot(xb, wk_ref[...].astype(jnp.bfloat16),
                                  preferred_element_type=jnp.float32),
                          cos_f, sin_f).astype(jnp.bfloat16)
        v_ref[...] = jnp.dot(xb, wv_ref[...].astype(jnp.bfloat16),
                             preferred_element_type=jnp.float32)

        def ctx_chunk(rc):
            ro = rc * CH
            xq = xb_ref[pl.ds(ro, CH), :]
            q = rope(jnp.dot(xq, wqb_ref[...],
                             preferred_element_type=jnp.float32),
                     cos_ref[pl.ds(ro, CH), :], sin_ref[pl.ds(ro, CH), :])
            q = (q * SCALE).astype(jnp.bfloat16)
            parts = []
            for h in range(HQ):
                sl = pl.ds(h * DH, DH)
                s = lax.dot_general(q[:, h * DH:(h + 1) * DH], k_ref[:, sl],
                                    (((1,), (1,)), ((), ())),
                                    preferred_element_type=jnp.float32)
                w = jnp.exp(s)
                ctx = jnp.dot(w, v_ref[:, sl],
                              preferred_element_type=jnp.float32)
                parts.append(ctx / jnp.sum(w, axis=-1, keepdims=True))
            return jnp.concatenate(parts, axis=1).astype(jnp.bfloat16)

        def proj_r(ctx):
            return jnp.dot(ctx, wob_ref[:, :HD],
                           preferred_element_type=jnp.float32
                           ).astype(jnp.bfloat16)

        def proj_l(ctx):
            return jnp.dot(ctx, wob_ref[:, HD:],
                           preferred_element_type=jnp.float32
                           ).astype(jnp.bfloat16)

        def add_bf(a, b):
            return (a.astype(jnp.float32) + b.astype(jnp.float32)
                    ).astype(jnp.bfloat16)

        def copy(src, dst, send, recv, slot, dev):
            return pltpu.make_async_remote_copy(
                src_ref=src, dst_ref=dst.at[slot],
                send_sem=send.at[slot], recv_sem=recv.at[slot],
                device_id=(dev,), device_id_type=pl.DeviceIdType.MESH)

        ctx0 = ctx_chunk(my)
        pr_ref[0] = proj_r(ctx0)
        pl_ref[0] = proj_l(ctx0)
        rsr0 = copy(pr_ref.at[0], rsr_ref, rsr_send, rsr_recv, 0, right)
        rsr0.start()
        rsl0 = copy(pl_ref.at[0], rsl_ref, rsl_send, rsl_recv, 0, left)
        rsl0.start()

        ctx_ref[0] = ctx_chunk(lax.rem(my + 3, N_DEV))
        pr_ref[1] = proj_r(ctx_ref[0])
        ctx_ref[1] = ctx_chunk(lax.rem(my + 1, N_DEV))
        pl_ref[1] = proj_l(ctx_ref[1])
        rsr0.wait_recv()
        rsr_ref[0] = add_bf(rsr_ref[0], pr_ref[1])
        rsr1 = copy(rsr_ref.at[0], rsr_ref, rsr_send, rsr_recv, 1, right)
        rsr1.start()
        rsl0.wait_recv()
        rsl_ref[0] = add_bf(rsl_ref[0], pl_ref[1])
        rsl1 = copy(rsl_ref.at[0], rsl_ref, rsl_send, rsl_recv, 1, left)
        rsl1.start()

        ctx2 = ctx_chunk(lax.rem(my + 2, N_DEV))
        pr_ref[2] = proj_r(ctx2)
        pl_ref[2] = proj_l(ctx2)
        rsr1.wait_recv()
        rsr_ref[1] = add_bf(rsr_ref[1], pr_ref[2])
        rsr2 = copy(rsr_ref.at[1], rsr_ref, rsr_send, rsr_recv, 2, right)
        rsr2.start()
        rsl1.wait_recv()
        rsl_ref[1] = add_bf(rsl_ref[1], pl_ref[2])
        rsl2 = copy(rsl_ref.at[1], rsl_ref, rsl_send, rsl_recv, 2, left)
        rsl2.start()

        pr_ref[3] = proj_r(ctx_ref[1])
        pl_ref[3] = proj_l(ctx_ref[0])
        rsr2.wait_recv()
        pr_ref[3] = add_bf(rsr_ref[2], pr_ref[3])
        agr0 = copy(pr_ref.at[3], agr_ref, agr_send, agr_recv, 0, right)
        agr0.start()
        rsl2.wait_recv()
        pl_ref[3] = add_bf(rsl_ref[2], pl_ref[3])
        agl0 = copy(pl_ref.at[3], agl_ref, agl_send, agl_recv, 0, left)
        agl0.start()
        out_ref[0, pl.ds(lax.rem(my + 1, N_DEV) * CH, CH), :HD] = (
            pr_ref[3].astype(jnp.float32))
        out_ref[0, pl.ds(lax.rem(my + 3, N_DEV) * CH, CH), HD:] = (
            pl_ref[3].astype(jnp.float32))

        agr0.wait_recv()
        agr1 = copy(agr_ref.at[0], agr_ref, agr_send, agr_recv, 1, right)
        agr1.start()
        out_ref[0, pl.ds(my * CH, CH), :HD] = agr_ref[0].astype(jnp.float32)
        agl0.wait_recv()
        agl1 = copy(agl_ref.at[0], agl_ref, agl_send, agl_recv, 1, left)
        agl1.start()
        out_ref[0, pl.ds(my * CH, CH), HD:] = agl_ref[0].astype(jnp.float32)

        agr1.wait_recv()
        agr2 = copy(agr_ref.at[1], agr_ref, agr_send, agr_recv, 2, right)
        agr2.start()
        out_ref[0, pl.ds(lax.rem(my + 3, N_DEV) * CH, CH), :HD] = (
            agr_ref[1].astype(jnp.float32))
        agl1.wait_recv()
        agl2 = copy(agl_ref.at[1], agl_ref, agl_send, agl_recv, 2, left)
        agl2.start()
        out_ref[0, pl.ds(lax.rem(my + 1, N_DEV) * CH, CH), HD:] = (
            agl_ref[1].astype(jnp.float32))

        agr2.wait_recv()
        out_ref[0, pl.ds(lax.rem(my + 2, N_DEV) * CH, CH), :HD] = (
            agr_ref[2].astype(jnp.float32))
        agl2.wait_recv()
        out_ref[0, pl.ds(lax.rem(my + 2, N_DEV) * CH, CH), HD:] = (
            agl_ref[2].astype(jnp.float32))

        for d in (rsr0, rsr1, rsr2, rsl0, rsl1, rsl2,
                  agr0, agr1, agr2, agl0, agl1, agl2):
            d.wait_send()

    cos = jnp.asarray(_COS)
    sin = jnp.asarray(_SIN)
    dma3 = pltpu.SemaphoreType.DMA((N_DEV - 1,))
    return pl.pallas_call(
        body,
        out_shape=jax.ShapeDtypeStruct((1, SQ, D), jnp.float32),
        in_specs=[pl.BlockSpec(memory_space=pltpu.VMEM)] * 7,
        out_specs=pl.BlockSpec(memory_space=pltpu.VMEM),
        scratch_shapes=[
            pltpu.VMEM((SQ, D), jnp.bfloat16),
            pltpu.VMEM((D, D), jnp.bfloat16),
            pltpu.VMEM((D, D), jnp.bfloat16),
            pltpu.VMEM((SQ, D), jnp.bfloat16),
            pltpu.VMEM((SQ, D), jnp.float32),
            pltpu.VMEM((2, CH, D), jnp.bfloat16),
            pltpu.VMEM((N_DEV, CH, HD), jnp.bfloat16),
            pltpu.VMEM((N_DEV, CH, HD), jnp.bfloat16),
            pltpu.VMEM((N_DEV - 1, CH, HD), jnp.bfloat16),
            pltpu.VMEM((N_DEV - 1, CH, HD), jnp.bfloat16),
            pltpu.VMEM((N_DEV - 1, CH, HD), jnp.bfloat16),
            pltpu.VMEM((N_DEV - 1, CH, HD), jnp.bfloat16),
            dma3, dma3, dma3, dma3,
            dma3, dma3, dma3, dma3,
        ],
        compiler_params=pltpu.CompilerParams(
            collective_id=0, vmem_limit_bytes=100 * 1024 * 1024),
    )(x, Wq, Wk, Wv, Wo, cos, sin)
